# baseline (speedup 1.0000x reference)
import functools
import jax
import jax.numpy as jnp
from jax import lax
from jax.experimental import pallas as pl
from jax.experimental.pallas import tpu as pltpu
from jax.experimental.pallas import tpu_sc as plsc

N = 10000
E = 320000
D_IN = 128
D_MP = 64
K = 10
DELTA = 0.311
EPS = 1e-3
TV_COEFF = 0.785
BAL_COEFF = 0.514

NPAD = 10240
NW = 32
EPW = E // NW
CHUNK = 400
NCHUNK = EPW // CHUNK
CHUNK_E = 80
NCHUNK_E = EPW // CHUNK_E
ROWS_PER_TILE = NPAD // 16


def _mm_body(x_ref, w_ref, o_ref):
    o_ref[...] = jnp.dot(x_ref[...], w_ref[...],
                         preferred_element_type=jnp.float32,
                         precision=lax.Precision.HIGHEST)


def _matmul(x, w):
    bn = 2000
    return pl.pallas_call(
        _mm_body,
        grid=(N // bn,),
        in_specs=[pl.BlockSpec((bn, D_IN), lambda i: (i, 0)),
                  pl.BlockSpec((D_IN, D_MP), lambda i: (0, 0))],
        out_specs=pl.BlockSpec((bn, D_MP), lambda i: (i, 0)),
        out_shape=jax.ShapeDtypeStruct((N, D_MP), jnp.float32),
    )(x, w)


def _hsum_splat(v):
    for sh in (8, 4, 2, 1):
        idx = jnp.arange(16, dtype=jnp.int32) ^ sh
        v = v + jnp.take(v, idx, mode="promise_in_bounds")
    return v


def _edge_body(xp_hbm, src_hbm, dst_hbm, ew_hbm, scat_hbm,
               table, ra0, ra1, rb0, rb1, msg, tbuf, si0, si1, di0, di1,
               ew0, ew1, ga0, ga1, gb0, gb1):
    cid = lax.axis_index("c")
    sid = lax.axis_index("s")
    wid = cid * 16 + sid
    r0 = sid * ROWS_PER_TILE
    ras, rbs = (ra0, ra1), (rb0, rb1)
    sis, dis, ews = (si0, si1), (di0, di1), (ew0, ew1)
    gas, gbs = (ga0, ga1), (gb0, gb1)

    def _z(i, _):
        zz = jnp.zeros((16,), jnp.float32)
        for dd in range(4):
            msg[i, pl.ds(dd * 16, 16)] = zz
        return 0
    lax.fori_loop(0, CHUNK_E, _z, 0)
    pltpu.sync_copy(msg, table.at[pl.ds(r0, CHUNK_E)])
    pltpu.sync_copy(msg.at[pl.ds(0, ROWS_PER_TILE - CHUNK_E)],
                    table.at[pl.ds(r0 + CHUNK_E, ROWS_PER_TILE - CHUNK_E)])
    plsc.subcore_barrier()

    def _load_and_gather(c, p):
        base = wid * EPW + c * CHUNK_E
        pltpu.sync_copy(src_hbm.at[pl.ds(base, CHUNK_E)], sis[p])
        pltpu.sync_copy(dst_hbm.at[pl.ds(base, CHUNK_E)], dis[p])
        pltpu.sync_copy(ew_hbm.at[pl.ds(base, CHUNK_E)], ews[p])
        pltpu.async_copy(xp_hbm.at[sis[p]], ras[p], gas[p])
        pltpu.async_copy(xp_hbm.at[dis[p]], rbs[p], gbs[p])

    def _wait_gather(p):
        pltpu.make_async_copy(xp_hbm.at[sis[p]], ras[p], gas[p]).wait()
        pltpu.make_async_copy(xp_hbm.at[dis[p]], rbs[p], gbs[p]).wait()

    def _compute_scatter(p):
        ra, rb, ewv = ras[p], rbs[p], ews[p]

        iota = lax.iota(jnp.int32, 16)
        shuf = [iota ^ sh for sh in (8, 4, 2, 1)]

        def _edges(i16, _):
            ebase = jnp.broadcast_to(i16 * 16, (16,)).astype(jnp.int32)
            for j in range(16):
                e = i16 * 16 + j
                rows = jnp.full((16,), j, jnp.int32)
                d = []
                t = None
                for dd in range(4):
                    a = ra[e, pl.ds(dd * 16, 16)]
                    b = rb[e, pl.ds(dd * 16, 16)]
                    dif = a - b
                    d.append(dif)
                    ad = jnp.abs(dif)
                    t = ad if t is None else t + ad
                for r, sh in enumerate(shuf):
                    tbuf[(r % 2) * 16 + j] = t
                    rows_r = rows + (r % 2) * 16
                    t = t + plsc.load_gather(tbuf, [rows_r, sh])
                gs = plsc.load_gather(ewv, [ebase + j])
                g = gs / jnp.maximum(t, EPS)
                for dd in range(4):
                    msg[e, pl.ds(dd * 16, 16)] = g * d[dd]
            return 0
        lax.fori_loop(0, CHUNK_E // 16, _edges, 0)
        pltpu.sync_copy(msg, table.at[sis[p]], add=True)

    _load_and_gather(0, 0)

    def _body(c2, _):
        c = 2 * c2
        _load_and_gather(c + 1, 1)
        _wait_gather(0)
        _compute_scatter(0)
        _load_and_gather(c + 2, 0)
        _wait_gather(1)
        _compute_scatter(1)
        return 0
    lax.fori_loop(0, (NCHUNK_E - 1) // 2, _body, 0)
    _wait_gather(0)
    _compute_scatter(0)

    plsc.subcore_barrier()
    pltpu.sync_copy(table.at[pl.ds(r0, ROWS_PER_TILE)],
                    scat_hbm.at[cid, pl.ds(r0, ROWS_PER_TILE)])


def _edge_pass(xp, src, dst, ew):
    mesh = plsc.VectorSubcoreMesh(core_axis_name="c", subcore_axis_name="s")
    f = pl.kernel(
        _edge_body,
        out_type=jax.ShapeDtypeStruct((2, NPAD, D_MP), jnp.float32),
        mesh=mesh,
        compiler_params=pltpu.CompilerParams(needs_layout_passes=False, use_tc_tiling_on_sc=False),
        scratch_types=(
            [pltpu.VMEM_SHARED((NPAD, D_MP), jnp.float32)]
            + [pltpu.VMEM((CHUNK_E, D_MP), jnp.float32)] * 5
            + [pltpu.VMEM((32, 16), jnp.float32)]
            + [pltpu.VMEM((CHUNK_E,), jnp.int32)] * 4
            + [pltpu.VMEM((CHUNK_E,), jnp.float32)] * 2
            + [pltpu.SemaphoreType.DMA] * 4
        ),
    )
    return f(xp, src, dst, ew)


def _act_body(xp_ref, scat_ref, b1_ref, wp_ref, bp_ref, s_ref, sp_ref):
    out = xp_ref[...] - DELTA * (scat_ref[0] + scat_ref[1])
    h = jnp.maximum(out + b1_ref[...], 0.0)
    logits = jnp.dot(h, wp_ref[...], preferred_element_type=jnp.float32,
                     precision=lax.Precision.HIGHEST) + bp_ref[...]
    m = jnp.max(logits, axis=-1, keepdims=True)
    ex = jnp.exp(logits - m)
    s = ex / jnp.sum(ex, axis=-1, keepdims=True)
    s_ref[...] = s
    sp_ref[...] = jnp.concatenate(
        [s, jnp.zeros((s.shape[0], 16 - K), jnp.float32)], axis=1)


def _act(xp, scat, b1, wp, bp):
    bn = 2000
    return pl.pallas_call(
        _act_body,
        grid=(N // bn,),
        in_specs=[pl.BlockSpec((bn, D_MP), lambda i: (i, 0)),
                  pl.BlockSpec((2, bn, D_MP), lambda i: (0, i, 0)),
                  pl.BlockSpec((1, D_MP), lambda i: (0, 0)),
                  pl.BlockSpec((D_MP, K), lambda i: (0, 0)),
                  pl.BlockSpec((1, K), lambda i: (0, 0))],
        out_specs=[pl.BlockSpec((bn, K), lambda i: (i, 0)),
                   pl.BlockSpec((bn, 16), lambda i: (i, 0))],
        out_shape=[jax.ShapeDtypeStruct((N, K), jnp.float32),
                   jax.ShapeDtypeStruct((N, 16), jnp.float32)],
    )(xp, scat, b1, wp, bp)


def _tv_body(sp_hbm, src_hbm, dst_hbm, ew_hbm, tvp_hbm,
             ra0, ra1, rb0, rb1, si0, si1, di0, di1, ew0, ew1, accv,
             ga0, ga1, gb0, gb1):
    cid = lax.axis_index("c")
    sid = lax.axis_index("s")
    wid = cid * 16 + sid
    ras, rbs = (ra0, ra1), (rb0, rb1)
    sis, dis, ews = (si0, si1), (di0, di1), (ew0, ew1)
    gas, gbs = (ga0, ga1), (gb0, gb1)

    def _load_and_gather(c, p):
        base = wid * EPW + c * CHUNK
        pltpu.sync_copy(src_hbm.at[pl.ds(base, CHUNK)], sis[p])
        pltpu.sync_copy(dst_hbm.at[pl.ds(base, CHUNK)], dis[p])
        pltpu.sync_copy(ew_hbm.at[pl.ds(base, CHUNK)], ews[p])
        pltpu.async_copy(sp_hbm.at[sis[p]], ras[p], gas[p])
        pltpu.async_copy(sp_hbm.at[dis[p]], rbs[p], gbs[p])

    def _wait_gather(p):
        pltpu.make_async_copy(sp_hbm.at[sis[p]], ras[p], gas[p]).wait()
        pltpu.make_async_copy(sp_hbm.at[dis[p]], rbs[p], gbs[p]).wait()

    def _accum(p, acc):
        ra, rb, ewv = ras[p], rbs[p], ews[p]

        def _edges(i16, acc4):
            a0, a1, a2, a3 = acc4
            ew16 = ewv[pl.ds(i16 * 16, 16)]
            for j in range(0, 16, 4):
                e = i16 * 16 + j
                a0 = a0 + ew16[j] * jnp.abs(ra[e] - rb[e])
                a1 = a1 + ew16[j + 1] * jnp.abs(ra[e + 1] - rb[e + 1])
                a2 = a2 + ew16[j + 2] * jnp.abs(ra[e + 2] - rb[e + 2])
                a3 = a3 + ew16[j + 3] * jnp.abs(ra[e + 3] - rb[e + 3])
            return (a0, a1, a2, a3)
        z = jnp.zeros((16,), jnp.float32)
        a0, a1, a2, a3 = lax.fori_loop(0, CHUNK // 16, _edges, (z, z, z, z))
        return acc + (a0 + a1) + (a2 + a3)

    _load_and_gather(0, 0)

    def _body(c2, acc):
        c = 2 * c2
        _load_and_gather(c + 1, 1)
        _wait_gather(0)
        acc = _accum(0, acc)
        _load_and_gather(c + 2, 0)
        _wait_gather(1)
        acc = _accum(1, acc)
        return acc
    acc = lax.fori_loop(0, (NCHUNK - 1) // 2, _body,
                        jnp.zeros((16,), jnp.float32))
    _wait_gather(0)
    acc = _accum(0, acc)
    accv[...] = acc
    pltpu.sync_copy(accv, tvp_hbm.at[wid])


def _tv_pass(sp, src, dst, ew):
    mesh = plsc.VectorSubcoreMesh(core_axis_name="c", subcore_axis_name="s")
    f = pl.kernel(
        _tv_body,
        out_type=jax.ShapeDtypeStruct((NW, 16), jnp.float32),
        mesh=mesh,
        compiler_params=pltpu.CompilerParams(needs_layout_passes=False, use_tc_tiling_on_sc=False),
        scratch_types=(
            [pltpu.VMEM((CHUNK, 16), jnp.float32)] * 4
            + [pltpu.VMEM((CHUNK,), jnp.int32)] * 4
            + [pltpu.VMEM((CHUNK,), jnp.float32)] * 2
            + [pltpu.VMEM((16,), jnp.float32)]
            + [pltpu.SemaphoreType.DMA] * 4
        ),
    )
    return f(sp, src, dst, ew)


def _loss_body(s_ref, tvp_ref, o_ref):
    s = s_ref[...]
    idx = N // K

    def _bisect(_, lohi):
        lo, hi = lohi
        mid = (lo + hi) >> 1
        v = lax.bitcast_convert_type(mid, jnp.float32)
        cnt = jnp.sum((s >= v).astype(jnp.int32), axis=0, keepdims=True)
        ok = cnt >= (idx + 1)
        return (jnp.where(ok, mid, lo), jnp.where(ok, hi, mid))
    lo0 = jnp.zeros((1, K), jnp.int32)
    hi0 = jnp.full((1, K), jnp.int32(0x3F800001))
    lo, _ = lax.fori_loop(0, 32, _bisect, (lo0, hi0))
    quant = lax.bitcast_convert_type(lo, jnp.float32)
    temp = s - quant
    asym = jnp.sum(jnp.where(temp >= 0, (K - 1.0) * temp, -temp))
    bal = BAL_COEFF * (1.0 / (N * (K - 1))) * (N * (K - 1) - asym)
    tv = TV_COEFF * jnp.sum(tvp_ref[...]) / (2.0 * E)
    o_ref[...] = jnp.broadcast_to(tv + bal, (1, 1))


def _loss(s, tvp):
    return pl.pallas_call(
        _loss_body,
        in_specs=[pl.BlockSpec((N, K), lambda: (0, 0)),
                  pl.BlockSpec((NW, 16), lambda: (0, 0))],
        out_specs=pl.BlockSpec((1, 1), lambda: (0, 0)),
        out_shape=jax.ShapeDtypeStruct((1, 1), jnp.float32),
    )(s, tvp)


def kernel(x, edge_index, edge_weight, W1, b1, Wp, bp):
    src = edge_index[0].astype(jnp.int32)
    dst = edge_index[1].astype(jnp.int32)
    xp = _matmul(x, W1)
    scat = _edge_pass(xp, src, dst, edge_weight)
    s, sp = _act(xp, scat, b1.reshape(1, D_MP), Wp, bp.reshape(1, K))
    tvp = _tv_pass(sp, src, dst, edge_weight)
    aux = _loss(s, tvp)
    return s, aux[0, 0]


# speedup vs baseline: 1.5753x; 1.5753x over previous
import functools
import jax
import jax.numpy as jnp
from jax import lax
from jax.experimental import pallas as pl
from jax.experimental.pallas import tpu as pltpu
from jax.experimental.pallas import tpu_sc as plsc

N = 10000
E = 320000
D_IN = 128
D_MP = 64
K = 10
DELTA = 0.311
EPS = 1e-3
TV_COEFF = 0.785
BAL_COEFF = 0.514

NPAD = 10240
NW = 32
EPW = E // NW
CHUNK = 400
NCHUNK = EPW // CHUNK
CHUNK_E = 80
NCHUNK_E = EPW // CHUNK_E
ROWS_PER_TILE = NPAD // 16


def _mm_body(x_ref, w_ref, o_ref):
    o_ref[...] = jnp.dot(x_ref[...], w_ref[...],
                         preferred_element_type=jnp.float32,
                         precision=lax.Precision.HIGHEST)


def _matmul(x, w):
    bn = 2000
    return pl.pallas_call(
        _mm_body,
        grid=(N // bn,),
        in_specs=[pl.BlockSpec((bn, D_IN), lambda i: (i, 0)),
                  pl.BlockSpec((D_IN, D_MP), lambda i: (0, 0))],
        out_specs=pl.BlockSpec((bn, D_MP), lambda i: (i, 0)),
        out_shape=jax.ShapeDtypeStruct((N, D_MP), jnp.float32),
    )(x, w)


def _hsum_splat(v):
    for sh in (8, 4, 2, 1):
        idx = jnp.arange(16, dtype=jnp.int32) ^ sh
        v = v + jnp.take(v, idx, mode="promise_in_bounds")
    return v


def _edge_body(xp_hbm, src_hbm, dst_hbm, ew_hbm, scat_hbm,
               table, ra0, ra1, rb0, rb1, si0, si1, di0, di1,
               ew0, ew1, ga0, ga1, gb0, gb1):
    cid = lax.axis_index("c")
    sid = lax.axis_index("s")
    wid = cid * 16 + sid
    r0 = sid * ROWS_PER_TILE
    ras, rbs = (ra0, ra1), (rb0, rb1)
    sis, dis, ews = (si0, si1), (di0, di1), (ew0, ew1)
    gas, gbs = (ga0, ga1), (gb0, gb1)

    def _z(i, _):
        zz = jnp.zeros((16,), jnp.float32)
        for dd in range(4):
            ra0[i, pl.ds(dd * 16, 16)] = zz
        return 0
    lax.fori_loop(0, CHUNK_E, _z, 0)
    for k in range(ROWS_PER_TILE // CHUNK_E):
        pltpu.sync_copy(ra0, table.at[pl.ds(r0 + k * CHUNK_E, CHUNK_E)])
    plsc.subcore_barrier()

    def _load_and_gather(c, p):
        base = wid * EPW + c * CHUNK_E
        pltpu.sync_copy(src_hbm.at[pl.ds(base, CHUNK_E)], sis[p])
        pltpu.sync_copy(dst_hbm.at[pl.ds(base, CHUNK_E)], dis[p])
        pltpu.sync_copy(ew_hbm.at[pl.ds(base, CHUNK_E)], ews[p])
        pltpu.async_copy(xp_hbm.at[sis[p]], ras[p], gas[p])
        pltpu.async_copy(xp_hbm.at[dis[p]], rbs[p], gbs[p])

    def _wait_gather(p):
        pltpu.make_async_copy(xp_hbm.at[sis[p]], ras[p], gas[p]).wait()
        pltpu.make_async_copy(xp_hbm.at[dis[p]], rbs[p], gbs[p]).wait()

    def _compute_scatter(p):
        ra, rb, ewv = ras[p], rbs[p], ews[p]

        def _edges(i16, _):
            ew16 = ewv[pl.ds(i16 * 16, 16)]
            for j in range(16):
                e = i16 * 16 + j
                d = []
                t = None
                for dd in range(4):
                    a = ra[e, pl.ds(dd * 16, 16)]
                    b = rb[e, pl.ds(dd * 16, 16)]
                    dif = a - b
                    d.append(dif)
                    ad = jnp.abs(dif)
                    t = ad if t is None else t + ad
                l1 = jnp.broadcast_to(jnp.sum(t), (16,))
                g = jnp.broadcast_to(ew16[j], (16,)) / jnp.maximum(l1, EPS)
                for dd in range(4):
                    ra[e, pl.ds(dd * 16, 16)] = g * d[dd]
            return 0
        lax.fori_loop(0, CHUNK_E // 16, _edges, 0)
        pltpu.sync_copy(ra, table.at[sis[p]], add=True)

    _load_and_gather(0, 0)

    def _body(c2, _):
        c = 2 * c2
        _load_and_gather(c + 1, 1)
        _wait_gather(0)
        _compute_scatter(0)
        _load_and_gather(c + 2, 0)
        _wait_gather(1)
        _compute_scatter(1)
        return 0
    lax.fori_loop(0, (NCHUNK_E - 1) // 2, _body, 0)
    _wait_gather(0)
    _compute_scatter(0)

    plsc.subcore_barrier()
    pltpu.sync_copy(table.at[pl.ds(r0, ROWS_PER_TILE)],
                    scat_hbm.at[cid, pl.ds(r0, ROWS_PER_TILE)])


def _edge_pass(xp, src, dst, ew):
    mesh = plsc.VectorSubcoreMesh(core_axis_name="c", subcore_axis_name="s")
    f = pl.kernel(
        _edge_body,
        out_type=jax.ShapeDtypeStruct((2, NPAD, D_MP), jnp.float32),
        mesh=mesh,
        compiler_params=pltpu.CompilerParams(needs_layout_passes=False, use_tc_tiling_on_sc=False),
        scratch_types=(
            [pltpu.VMEM_SHARED((NPAD, D_MP), jnp.float32)]
            + [pltpu.VMEM((CHUNK_E, D_MP), jnp.float32)] * 4
            + [pltpu.VMEM((CHUNK_E,), jnp.int32)] * 4
            + [pltpu.VMEM((CHUNK_E,), jnp.float32)] * 2
            + [pltpu.SemaphoreType.DMA] * 4
        ),
    )
    return f(xp, src, dst, ew)


def _act_body(xp_ref, scat_ref, b1_ref, wp_ref, bp_ref,
              s_ref, sp_ref, bal_ref):
    out = xp_ref[...] - DELTA * (scat_ref[0, :N] + scat_ref[1, :N])
    h = jnp.maximum(out + b1_ref[...], 0.0)
    logits = jnp.dot(h, wp_ref[...], preferred_element_type=jnp.float32,
                     precision=lax.Precision.HIGHEST) + bp_ref[...]
    m = jnp.max(logits, axis=-1, keepdims=True)
    ex = jnp.exp(logits - m)
    s = ex / jnp.sum(ex, axis=-1, keepdims=True)
    s_ref[...] = s
    sp_ref[...] = jnp.concatenate(
        [s, jnp.zeros((s.shape[0], 16 - K), jnp.float32)], axis=1)
    idx = N // K

    def _bisect(_, lohi):
        lo, hi = lohi
        mid = (lo + hi) >> 1
        v = lax.bitcast_convert_type(mid, jnp.float32)
        cnt = jnp.sum((s >= v).astype(jnp.int32), axis=0, keepdims=True)
        ok = cnt >= (idx + 1)
        return (jnp.where(ok, mid, lo), jnp.where(ok, hi, mid))
    lo0 = jnp.zeros((1, K), jnp.int32)
    hi0 = jnp.full((1, K), jnp.int32(0x3F800001))
    lo, _ = lax.fori_loop(0, 32, _bisect, (lo0, hi0))
    quant = lax.bitcast_convert_type(lo, jnp.float32)
    temp = s - quant
    asym = jnp.sum(jnp.where(temp >= 0, (K - 1.0) * temp, -temp))
    bal = BAL_COEFF * (1.0 / (N * (K - 1))) * (N * (K - 1) - asym)
    bal_ref[...] = jnp.broadcast_to(bal, (1, 1))


def _act(xp, scat, b1, wp, bp):
    return pl.pallas_call(
        _act_body,
        out_shape=[jax.ShapeDtypeStruct((N, K), jnp.float32),
                   jax.ShapeDtypeStruct((N, 16), jnp.float32),
                   jax.ShapeDtypeStruct((1, 1), jnp.float32)],
    )(xp, scat, b1, wp, bp)


def _tv_body(sp_hbm, src_hbm, dst_hbm, ew_hbm, tvp_hbm,
             ra0, ra1, rb0, rb1, si0, si1, di0, di1, ew0, ew1, accv,
             ga0, ga1, gb0, gb1):
    cid = lax.axis_index("c")
    sid = lax.axis_index("s")
    wid = cid * 16 + sid
    ras, rbs = (ra0, ra1), (rb0, rb1)
    sis, dis, ews = (si0, si1), (di0, di1), (ew0, ew1)
    gas, gbs = (ga0, ga1), (gb0, gb1)

    def _load_and_gather(c, p):
        base = wid * EPW + c * CHUNK
        pltpu.sync_copy(src_hbm.at[pl.ds(base, CHUNK)], sis[p])
        pltpu.sync_copy(dst_hbm.at[pl.ds(base, CHUNK)], dis[p])
        pltpu.sync_copy(ew_hbm.at[pl.ds(base, CHUNK)], ews[p])
        pltpu.async_copy(sp_hbm.at[sis[p]], ras[p], gas[p])
        pltpu.async_copy(sp_hbm.at[dis[p]], rbs[p], gbs[p])

    def _wait_gather(p):
        pltpu.make_async_copy(sp_hbm.at[sis[p]], ras[p], gas[p]).wait()
        pltpu.make_async_copy(sp_hbm.at[dis[p]], rbs[p], gbs[p]).wait()

    def _accum(p, acc):
        ra, rb, ewv = ras[p], rbs[p], ews[p]

        def _edges(i16, acc4):
            a0, a1, a2, a3 = acc4
            ew16 = ewv[pl.ds(i16 * 16, 16)]
            for j in range(0, 16, 4):
                e = i16 * 16 + j
                a0 = a0 + ew16[j] * jnp.abs(ra[e] - rb[e])
                a1 = a1 + ew16[j + 1] * jnp.abs(ra[e + 1] - rb[e + 1])
                a2 = a2 + ew16[j + 2] * jnp.abs(ra[e + 2] - rb[e + 2])
                a3 = a3 + ew16[j + 3] * jnp.abs(ra[e + 3] - rb[e + 3])
            return (a0, a1, a2, a3)
        z = jnp.zeros((16,), jnp.float32)
        a0, a1, a2, a3 = lax.fori_loop(0, CHUNK // 16, _edges, (z, z, z, z))
        return acc + (a0 + a1) + (a2 + a3)

    _load_and_gather(0, 0)

    def _body(c2, acc):
        c = 2 * c2
        _load_and_gather(c + 1, 1)
        _wait_gather(0)
        acc = _accum(0, acc)
        _load_and_gather(c + 2, 0)
        _wait_gather(1)
        acc = _accum(1, acc)
        return acc
    acc = lax.fori_loop(0, (NCHUNK - 1) // 2, _body,
                        jnp.zeros((16,), jnp.float32))
    _wait_gather(0)
    acc = _accum(0, acc)
    accv[...] = acc
    pltpu.sync_copy(accv, tvp_hbm.at[wid])


def _tv_pass(sp, src, dst, ew):
    mesh = plsc.VectorSubcoreMesh(core_axis_name="c", subcore_axis_name="s")
    f = pl.kernel(
        _tv_body,
        out_type=jax.ShapeDtypeStruct((NW, 16), jnp.float32),
        mesh=mesh,
        compiler_params=pltpu.CompilerParams(needs_layout_passes=False, use_tc_tiling_on_sc=False),
        scratch_types=(
            [pltpu.VMEM((CHUNK, 16), jnp.float32)] * 4
            + [pltpu.VMEM((CHUNK,), jnp.int32)] * 4
            + [pltpu.VMEM((CHUNK,), jnp.float32)] * 2
            + [pltpu.VMEM((16,), jnp.float32)]
            + [pltpu.SemaphoreType.DMA] * 4
        ),
    )
    return f(sp, src, dst, ew)


def kernel(x, edge_index, edge_weight, W1, b1, Wp, bp):
    src = edge_index[0].astype(jnp.int32)
    dst = edge_index[1].astype(jnp.int32)
    xp = _matmul(x, W1)
    scat = _edge_pass(xp, src, dst, edge_weight)
    s, sp, bal = _act(xp, scat, b1.reshape(1, D_MP), Wp, bp.reshape(1, K))
    tvp = _tv_pass(sp, src, dst, edge_weight)
    aux = TV_COEFF * jnp.sum(tvp) / (2.0 * E) + bal[0, 0]
    return s, aux


# speedup vs baseline: 1.9975x; 1.2680x over previous
import functools
import jax
import jax.numpy as jnp
from jax import lax
from jax.experimental import pallas as pl
from jax.experimental.pallas import tpu as pltpu
from jax.experimental.pallas import tpu_sc as plsc

N = 10000
E = 320000
D_IN = 128
D_MP = 64
K = 10
DELTA = 0.311
EPS = 1e-3
TV_COEFF = 0.785
BAL_COEFF = 0.514

NPAD = 10240
NW = 32
EPW = E // NW
CHUNK = 400
NCHUNK = EPW // CHUNK
CHUNK_E = 80
NCHUNK_E = EPW // CHUNK_E
ROWS_PER_TILE = NPAD // 16


def _mm_body(x_ref, w_ref, o_ref):
    o_ref[...] = jnp.dot(x_ref[...], w_ref[...],
                         preferred_element_type=jnp.float32,
                         precision=lax.Precision.HIGHEST)


def _matmul(x, w):
    bn = 2000
    return pl.pallas_call(
        _mm_body,
        grid=(N // bn,),
        in_specs=[pl.BlockSpec((bn, D_IN), lambda i: (i, 0)),
                  pl.BlockSpec((D_IN, D_MP), lambda i: (0, 0))],
        out_specs=pl.BlockSpec((bn, D_MP), lambda i: (i, 0)),
        out_shape=jax.ShapeDtypeStruct((N, D_MP), jnp.float32),
    )(x, w)


def _hsum_splat(v):
    for sh in (8, 4, 2, 1):
        idx = jnp.arange(16, dtype=jnp.int32) ^ sh
        v = v + jnp.take(v, idx, mode="promise_in_bounds")
    return v


def _edge_body(xp_hbm, src_hbm, dst_hbm, ew_hbm, scat_hbm,
               table, ra0, ra1, rb0, rb1, si0, si1, di0, di1,
               ew0, ew1, ga0, ga1, gb0, gb1):
    cid = lax.axis_index("c")
    sid = lax.axis_index("s")
    wid = cid * 16 + sid
    r0 = sid * ROWS_PER_TILE
    ras, rbs = (ra0, ra1), (rb0, rb1)
    sis, dis, ews = (si0, si1), (di0, di1), (ew0, ew1)
    gas, gbs = (ga0, ga1), (gb0, gb1)

    def _z(i, _):
        zz = jnp.zeros((16,), jnp.float32)
        for dd in range(4):
            ra0[i, pl.ds(dd * 16, 16)] = zz
        return 0
    lax.fori_loop(0, CHUNK_E, _z, 0)
    for k in range(ROWS_PER_TILE // CHUNK_E):
        pltpu.sync_copy(ra0, table.at[pl.ds(r0 + k * CHUNK_E, CHUNK_E)])
    plsc.subcore_barrier()

    def _load_and_gather(c, p):
        base = wid * EPW + c * CHUNK_E
        pltpu.sync_copy(src_hbm.at[pl.ds(base, CHUNK_E)], sis[p])
        pltpu.sync_copy(dst_hbm.at[pl.ds(base, CHUNK_E)], dis[p])
        pltpu.sync_copy(ew_hbm.at[pl.ds(base, CHUNK_E)], ews[p])
        pltpu.async_copy(xp_hbm.at[sis[p]], ras[p], gas[p])
        pltpu.async_copy(xp_hbm.at[dis[p]], rbs[p], gbs[p])

    def _wait_gather(p):
        pltpu.make_async_copy(xp_hbm.at[sis[p]], ras[p], gas[p]).wait()
        pltpu.make_async_copy(xp_hbm.at[dis[p]], rbs[p], gbs[p]).wait()

    def _compute_scatter(p):
        ra, rb, ewv = ras[p], rbs[p], ews[p]

        @functools.partial(plsc.parallel_loop, 0, CHUNK_E // 16, unroll=2)
        def _edges(i16):
            ew16 = ewv[pl.ds(i16 * 16, 16)]
            for j in range(16):
                e = i16 * 16 + j
                d = []
                t = None
                for dd in range(4):
                    a = ra[e, pl.ds(dd * 16, 16)]
                    b = rb[e, pl.ds(dd * 16, 16)]
                    dif = a - b
                    d.append(dif)
                    ad = jnp.abs(dif)
                    t = ad if t is None else t + ad
                l1 = jnp.broadcast_to(jnp.sum(t), (16,))
                g = jnp.broadcast_to(ew16[j], (16,)) / jnp.maximum(l1, EPS)
                for dd in range(4):
                    ra[e, pl.ds(dd * 16, 16)] = g * d[dd]
        pltpu.sync_copy(ra, table.at[sis[p]], add=True)

    _load_and_gather(0, 0)

    def _body(c2, _):
        c = 2 * c2
        _load_and_gather(c + 1, 1)
        _wait_gather(0)
        _compute_scatter(0)
        _load_and_gather(c + 2, 0)
        _wait_gather(1)
        _compute_scatter(1)
        return 0
    lax.fori_loop(0, (NCHUNK_E - 1) // 2, _body, 0)
    _wait_gather(0)
    _compute_scatter(0)

    plsc.subcore_barrier()
    pltpu.sync_copy(table.at[pl.ds(r0, ROWS_PER_TILE)],
                    scat_hbm.at[cid, pl.ds(r0, ROWS_PER_TILE)])


def _edge_pass(xp, src, dst, ew):
    mesh = plsc.VectorSubcoreMesh(core_axis_name="c", subcore_axis_name="s")
    f = pl.kernel(
        _edge_body,
        out_type=jax.ShapeDtypeStruct((2, NPAD, D_MP), jnp.float32),
        mesh=mesh,
        compiler_params=pltpu.CompilerParams(needs_layout_passes=False, use_tc_tiling_on_sc=False),
        scratch_types=(
            [pltpu.VMEM_SHARED((NPAD, D_MP), jnp.float32)]
            + [pltpu.VMEM((CHUNK_E, D_MP), jnp.float32)] * 4
            + [pltpu.VMEM((CHUNK_E,), jnp.int32)] * 4
            + [pltpu.VMEM((CHUNK_E,), jnp.float32)] * 2
            + [pltpu.SemaphoreType.DMA] * 4
        ),
    )
    return f(xp, src, dst, ew)


def _act_body(xp_ref, scat_ref, b1_ref, wp_ref, bp_ref,
              s_ref, sp_ref, bal_ref):
    out = xp_ref[...] - DELTA * (scat_ref[0, :N] + scat_ref[1, :N])
    h = jnp.maximum(out + b1_ref[...], 0.0)
    logits = jnp.dot(h, wp_ref[...], preferred_element_type=jnp.float32,
                     precision=lax.Precision.HIGHEST) + bp_ref[...]
    m = jnp.max(logits, axis=-1, keepdims=True)
    ex = jnp.exp(logits - m)
    s = ex / jnp.sum(ex, axis=-1, keepdims=True)
    s_ref[...] = s
    sp_ref[...] = jnp.concatenate(
        [s, jnp.zeros((s.shape[0], 16 - K), jnp.float32)], axis=1)
    idx = N // K

    def _bisect(_, lohi):
        lo, hi = lohi
        mid = (lo + hi) >> 1
        v = lax.bitcast_convert_type(mid, jnp.float32)
        cnt = jnp.sum((s >= v).astype(jnp.int32), axis=0, keepdims=True)
        ok = cnt >= (idx + 1)
        return (jnp.where(ok, mid, lo), jnp.where(ok, hi, mid))
    lo0 = jnp.zeros((1, K), jnp.int32)
    hi0 = jnp.full((1, K), jnp.int32(0x3F800001))
    lo, _ = lax.fori_loop(0, 32, _bisect, (lo0, hi0))
    quant = lax.bitcast_convert_type(lo, jnp.float32)
    temp = s - quant
    asym = jnp.sum(jnp.where(temp >= 0, (K - 1.0) * temp, -temp))
    bal = BAL_COEFF * (1.0 / (N * (K - 1))) * (N * (K - 1) - asym)
    bal_ref[...] = jnp.broadcast_to(bal, (1, 1))


def _act(xp, scat, b1, wp, bp):
    return pl.pallas_call(
        _act_body,
        out_shape=[jax.ShapeDtypeStruct((N, K), jnp.float32),
                   jax.ShapeDtypeStruct((N, 16), jnp.float32),
                   jax.ShapeDtypeStruct((1, 1), jnp.float32)],
    )(xp, scat, b1, wp, bp)


def _tv_body(sp_hbm, src_hbm, dst_hbm, ew_hbm, tvp_hbm,
             ra0, ra1, rb0, rb1, si0, si1, di0, di1, ew0, ew1, accv,
             ga0, ga1, gb0, gb1):
    cid = lax.axis_index("c")
    sid = lax.axis_index("s")
    wid = cid * 16 + sid
    ras, rbs = (ra0, ra1), (rb0, rb1)
    sis, dis, ews = (si0, si1), (di0, di1), (ew0, ew1)
    gas, gbs = (ga0, ga1), (gb0, gb1)

    def _load_and_gather(c, p):
        base = wid * EPW + c * CHUNK
        pltpu.sync_copy(src_hbm.at[pl.ds(base, CHUNK)], sis[p])
        pltpu.sync_copy(dst_hbm.at[pl.ds(base, CHUNK)], dis[p])
        pltpu.sync_copy(ew_hbm.at[pl.ds(base, CHUNK)], ews[p])
        pltpu.async_copy(sp_hbm.at[sis[p]], ras[p], gas[p])
        pltpu.async_copy(sp_hbm.at[dis[p]], rbs[p], gbs[p])

    def _wait_gather(p):
        pltpu.make_async_copy(sp_hbm.at[sis[p]], ras[p], gas[p]).wait()
        pltpu.make_async_copy(sp_hbm.at[dis[p]], rbs[p], gbs[p]).wait()

    def _accum(p, acc):
        ra, rb, ewv = ras[p], rbs[p], ews[p]

        def _edges(i16, acc4):
            a0, a1, a2, a3 = acc4
            ew16 = ewv[pl.ds(i16 * 16, 16)]
            for j in range(0, 16, 4):
                e = i16 * 16 + j
                a0 = a0 + ew16[j] * jnp.abs(ra[e] - rb[e])
                a1 = a1 + ew16[j + 1] * jnp.abs(ra[e + 1] - rb[e + 1])
                a2 = a2 + ew16[j + 2] * jnp.abs(ra[e + 2] - rb[e + 2])
                a3 = a3 + ew16[j + 3] * jnp.abs(ra[e + 3] - rb[e + 3])
            return (a0, a1, a2, a3)
        z = jnp.zeros((16,), jnp.float32)
        a0, a1, a2, a3 = lax.fori_loop(0, CHUNK // 16, _edges, (z, z, z, z))
        return acc + (a0 + a1) + (a2 + a3)

    _load_and_gather(0, 0)

    def _body(c2, acc):
        c = 2 * c2
        _load_and_gather(c + 1, 1)
        _wait_gather(0)
        acc = _accum(0, acc)
        _load_and_gather(c + 2, 0)
        _wait_gather(1)
        acc = _accum(1, acc)
        return acc
    acc = lax.fori_loop(0, (NCHUNK - 1) // 2, _body,
                        jnp.zeros((16,), jnp.float32))
    _wait_gather(0)
    acc = _accum(0, acc)
    accv[...] = acc
    pltpu.sync_copy(accv, tvp_hbm.at[wid])


def _tv_pass(sp, src, dst, ew):
    mesh = plsc.VectorSubcoreMesh(core_axis_name="c", subcore_axis_name="s")
    f = pl.kernel(
        _tv_body,
        out_type=jax.ShapeDtypeStruct((NW, 16), jnp.float32),
        mesh=mesh,
        compiler_params=pltpu.CompilerParams(needs_layout_passes=False, use_tc_tiling_on_sc=False),
        scratch_types=(
            [pltpu.VMEM((CHUNK, 16), jnp.float32)] * 4
            + [pltpu.VMEM((CHUNK,), jnp.int32)] * 4
            + [pltpu.VMEM((CHUNK,), jnp.float32)] * 2
            + [pltpu.VMEM((16,), jnp.float32)]
            + [pltpu.SemaphoreType.DMA] * 4
        ),
    )
    return f(sp, src, dst, ew)


def kernel(x, edge_index, edge_weight, W1, b1, Wp, bp):
    src = edge_index[0].astype(jnp.int32)
    dst = edge_index[1].astype(jnp.int32)
    xp = _matmul(x, W1)
    scat = _edge_pass(xp, src, dst, edge_weight)
    s, sp, bal = _act(xp, scat, b1.reshape(1, D_MP), Wp, bp.reshape(1, K))
    tvp = _tv_pass(sp, src, dst, edge_weight)
    aux = TV_COEFF * jnp.sum(tvp) / (2.0 * E) + bal[0, 0]
    return s, aux


# speedup vs baseline: 1.9977x; 1.0001x over previous
import functools
import jax
import jax.numpy as jnp
from jax import lax
from jax.experimental import pallas as pl
from jax.experimental.pallas import tpu as pltpu
from jax.experimental.pallas import tpu_sc as plsc

N = 10000
E = 320000
D_IN = 128
D_MP = 64
K = 10
DELTA = 0.311
EPS = 1e-3
TV_COEFF = 0.785
BAL_COEFF = 0.514

NPAD = 10240
NW = 32
EPW = E // NW
CHUNK = 400
NCHUNK = EPW // CHUNK
CHUNK_E = 80
NCHUNK_E = EPW // CHUNK_E
ROWS_PER_TILE = NPAD // 16


def _mm_body(x_ref, w_ref, o_ref):
    o_ref[...] = jnp.dot(x_ref[...], w_ref[...],
                         preferred_element_type=jnp.float32,
                         precision=lax.Precision.HIGHEST)


def _matmul(x, w):
    bn = 2000
    return pl.pallas_call(
        _mm_body,
        grid=(N // bn,),
        in_specs=[pl.BlockSpec((bn, D_IN), lambda i: (i, 0)),
                  pl.BlockSpec((D_IN, D_MP), lambda i: (0, 0))],
        out_specs=pl.BlockSpec((bn, D_MP), lambda i: (i, 0)),
        out_shape=jax.ShapeDtypeStruct((N, D_MP), jnp.float32),
    )(x, w)


def _hsum_splat(v):
    for sh in (8, 4, 2, 1):
        idx = jnp.arange(16, dtype=jnp.int32) ^ sh
        v = v + jnp.take(v, idx, mode="promise_in_bounds")
    return v


def _edge_body(xp_hbm, src_hbm, dst_hbm, ew_hbm, scat_hbm,
               table, ra0, ra1, rb0, rb1, msg, si0, si1, di0, di1,
               ew0, ew1, ga0, ga1, gb0, gb1):
    cid = lax.axis_index("c")
    sid = lax.axis_index("s")
    wid = cid * 16 + sid
    r0 = sid * ROWS_PER_TILE
    ras, rbs = (ra0, ra1), (rb0, rb1)
    sis, dis, ews = (si0, si1), (di0, di1), (ew0, ew1)
    gas, gbs = (ga0, ga1), (gb0, gb1)

    def _z(i, _):
        zz = jnp.zeros((16,), jnp.float32)
        for dd in range(4):
            ra0[i, pl.ds(dd * 16, 16)] = zz
        return 0
    lax.fori_loop(0, CHUNK_E, _z, 0)
    for k in range(ROWS_PER_TILE // CHUNK_E):
        pltpu.sync_copy(ra0, table.at[pl.ds(r0 + k * CHUNK_E, CHUNK_E)])
    plsc.subcore_barrier()

    def _load_and_gather(c, p):
        base = wid * EPW + c * CHUNK_E
        pltpu.sync_copy(src_hbm.at[pl.ds(base, CHUNK_E)], sis[p])
        pltpu.sync_copy(dst_hbm.at[pl.ds(base, CHUNK_E)], dis[p])
        pltpu.sync_copy(ew_hbm.at[pl.ds(base, CHUNK_E)], ews[p])
        pltpu.async_copy(xp_hbm.at[sis[p]], ras[p], gas[p])
        pltpu.async_copy(xp_hbm.at[dis[p]], rbs[p], gbs[p])

    def _wait_gather(p):
        pltpu.make_async_copy(xp_hbm.at[sis[p]], ras[p], gas[p]).wait()
        pltpu.make_async_copy(xp_hbm.at[dis[p]], rbs[p], gbs[p]).wait()

    def _compute_scatter(p):
        ra, rb, ewv = ras[p], rbs[p], ews[p]

        @functools.partial(plsc.parallel_loop, 0, CHUNK_E // 16, unroll=5)
        def _edges(i16):
            ew16 = ewv[pl.ds(i16 * 16, 16)]
            for j in range(16):
                e = i16 * 16 + j
                d = []
                t = None
                for dd in range(4):
                    a = ra[e, pl.ds(dd * 16, 16)]
                    b = rb[e, pl.ds(dd * 16, 16)]
                    dif = a - b
                    d.append(dif)
                    ad = jnp.abs(dif)
                    t = ad if t is None else t + ad
                l1 = jnp.broadcast_to(jnp.sum(t), (16,))
                g = jnp.broadcast_to(ew16[j], (16,)) / jnp.maximum(l1, EPS)
                for dd in range(4):
                    msg[e, pl.ds(dd * 16, 16)] = g * d[dd]
        pltpu.sync_copy(msg, table.at[sis[p]], add=True)

    _load_and_gather(0, 0)

    def _body(c2, _):
        c = 2 * c2
        _load_and_gather(c + 1, 1)
        _wait_gather(0)
        _compute_scatter(0)
        _load_and_gather(c + 2, 0)
        _wait_gather(1)
        _compute_scatter(1)
        return 0
    lax.fori_loop(0, (NCHUNK_E - 1) // 2, _body, 0)
    _wait_gather(0)
    _compute_scatter(0)

    plsc.subcore_barrier()
    pltpu.sync_copy(table.at[pl.ds(r0, ROWS_PER_TILE)],
                    scat_hbm.at[cid, pl.ds(r0, ROWS_PER_TILE)])


def _edge_pass(xp, src, dst, ew):
    mesh = plsc.VectorSubcoreMesh(core_axis_name="c", subcore_axis_name="s")
    f = pl.kernel(
        _edge_body,
        out_type=jax.ShapeDtypeStruct((2, NPAD, D_MP), jnp.float32),
        mesh=mesh,
        compiler_params=pltpu.CompilerParams(needs_layout_passes=False, use_tc_tiling_on_sc=False),
        scratch_types=(
            [pltpu.VMEM_SHARED((NPAD, D_MP), jnp.float32)]
            + [pltpu.VMEM((CHUNK_E, D_MP), jnp.float32)] * 5
            + [pltpu.VMEM((CHUNK_E,), jnp.int32)] * 4
            + [pltpu.VMEM((CHUNK_E,), jnp.float32)] * 2
            + [pltpu.SemaphoreType.DMA] * 4
        ),
    )
    return f(xp, src, dst, ew)


def _act_body(xp_ref, scat_ref, b1_ref, wp_ref, bp_ref,
              s_ref, sp_ref, bal_ref):
    out = xp_ref[...] - DELTA * (scat_ref[0, :N] + scat_ref[1, :N])
    h = jnp.maximum(out + b1_ref[...], 0.0)
    logits = jnp.dot(h, wp_ref[...], preferred_element_type=jnp.float32,
                     precision=lax.Precision.HIGHEST) + bp_ref[...]
    m = jnp.max(logits, axis=-1, keepdims=True)
    ex = jnp.exp(logits - m)
    s = ex / jnp.sum(ex, axis=-1, keepdims=True)
    s_ref[...] = s
    sp_ref[...] = jnp.concatenate(
        [s, jnp.zeros((s.shape[0], 16 - K), jnp.float32)], axis=1)
    idx = N // K

    def _bisect(_, lohi):
        lo, hi = lohi
        mid = (lo + hi) >> 1
        v = lax.bitcast_convert_type(mid, jnp.float32)
        cnt = jnp.sum((s >= v).astype(jnp.int32), axis=0, keepdims=True)
        ok = cnt >= (idx + 1)
        return (jnp.where(ok, mid, lo), jnp.where(ok, hi, mid))
    lo0 = jnp.zeros((1, K), jnp.int32)
    hi0 = jnp.full((1, K), jnp.int32(0x3F800001))
    lo, _ = lax.fori_loop(0, 32, _bisect, (lo0, hi0))
    quant = lax.bitcast_convert_type(lo, jnp.float32)
    temp = s - quant
    asym = jnp.sum(jnp.where(temp >= 0, (K - 1.0) * temp, -temp))
    bal = BAL_COEFF * (1.0 / (N * (K - 1))) * (N * (K - 1) - asym)
    bal_ref[...] = jnp.broadcast_to(bal, (1, 1))


def _act(xp, scat, b1, wp, bp):
    return pl.pallas_call(
        _act_body,
        out_shape=[jax.ShapeDtypeStruct((N, K), jnp.float32),
                   jax.ShapeDtypeStruct((N, 16), jnp.float32),
                   jax.ShapeDtypeStruct((1, 1), jnp.float32)],
    )(xp, scat, b1, wp, bp)


def _tv_body(sp_hbm, src_hbm, dst_hbm, ew_hbm, tvp_hbm,
             ra0, ra1, rb0, rb1, si0, si1, di0, di1, ew0, ew1, accv,
             ga0, ga1, gb0, gb1):
    cid = lax.axis_index("c")
    sid = lax.axis_index("s")
    wid = cid * 16 + sid
    ras, rbs = (ra0, ra1), (rb0, rb1)
    sis, dis, ews = (si0, si1), (di0, di1), (ew0, ew1)
    gas, gbs = (ga0, ga1), (gb0, gb1)

    def _load_and_gather(c, p):
        base = wid * EPW + c * CHUNK
        pltpu.sync_copy(src_hbm.at[pl.ds(base, CHUNK)], sis[p])
        pltpu.sync_copy(dst_hbm.at[pl.ds(base, CHUNK)], dis[p])
        pltpu.sync_copy(ew_hbm.at[pl.ds(base, CHUNK)], ews[p])
        pltpu.async_copy(sp_hbm.at[sis[p]], ras[p], gas[p])
        pltpu.async_copy(sp_hbm.at[dis[p]], rbs[p], gbs[p])

    def _wait_gather(p):
        pltpu.make_async_copy(sp_hbm.at[sis[p]], ras[p], gas[p]).wait()
        pltpu.make_async_copy(sp_hbm.at[dis[p]], rbs[p], gbs[p]).wait()

    def _accum(p, acc):
        ra, rb, ewv = ras[p], rbs[p], ews[p]

        def _edges(i16, acc4):
            a0, a1, a2, a3 = acc4
            ew16 = ewv[pl.ds(i16 * 16, 16)]
            for j in range(0, 16, 4):
                e = i16 * 16 + j
                a0 = a0 + ew16[j] * jnp.abs(ra[e] - rb[e])
                a1 = a1 + ew16[j + 1] * jnp.abs(ra[e + 1] - rb[e + 1])
                a2 = a2 + ew16[j + 2] * jnp.abs(ra[e + 2] - rb[e + 2])
                a3 = a3 + ew16[j + 3] * jnp.abs(ra[e + 3] - rb[e + 3])
            return (a0, a1, a2, a3)
        z = jnp.zeros((16,), jnp.float32)
        a0, a1, a2, a3 = lax.fori_loop(0, CHUNK // 16, _edges, (z, z, z, z))
        return acc + (a0 + a1) + (a2 + a3)

    _load_and_gather(0, 0)

    def _body(c2, acc):
        c = 2 * c2
        _load_and_gather(c + 1, 1)
        _wait_gather(0)
        acc = _accum(0, acc)
        _load_and_gather(c + 2, 0)
        _wait_gather(1)
        acc = _accum(1, acc)
        return acc
    acc = lax.fori_loop(0, (NCHUNK - 1) // 2, _body,
                        jnp.zeros((16,), jnp.float32))
    _wait_gather(0)
    acc = _accum(0, acc)
    accv[...] = acc
    pltpu.sync_copy(accv, tvp_hbm.at[wid])


def _tv_pass(sp, src, dst, ew):
    mesh = plsc.VectorSubcoreMesh(core_axis_name="c", subcore_axis_name="s")
    f = pl.kernel(
        _tv_body,
        out_type=jax.ShapeDtypeStruct((NW, 16), jnp.float32),
        mesh=mesh,
        compiler_params=pltpu.CompilerParams(needs_layout_passes=False, use_tc_tiling_on_sc=False),
        scratch_types=(
            [pltpu.VMEM((CHUNK, 16), jnp.float32)] * 4
            + [pltpu.VMEM((CHUNK,), jnp.int32)] * 4
            + [pltpu.VMEM((CHUNK,), jnp.float32)] * 2
            + [pltpu.VMEM((16,), jnp.float32)]
            + [pltpu.SemaphoreType.DMA] * 4
        ),
    )
    return f(sp, src, dst, ew)


def kernel(x, edge_index, edge_weight, W1, b1, Wp, bp):
    src = edge_index[0].astype(jnp.int32)
    dst = edge_index[1].astype(jnp.int32)
    xp = _matmul(x, W1)
    scat = _edge_pass(xp, src, dst, edge_weight)
    s, sp, bal = _act(xp, scat, b1.reshape(1, D_MP), Wp, bp.reshape(1, K))
    tvp = _tv_pass(sp, src, dst, edge_weight)
    aux = TV_COEFF * jnp.sum(tvp) / (2.0 * E) + bal[0, 0]
    return s, aux
